# baseline (speedup 1.0000x reference)
import functools

import jax
import jax.numpy as jnp
from jax import lax
from jax.experimental import pallas as pl
from jax.experimental.pallas import tpu as pltpu
from jax.experimental.pallas import tpu_sc as plsc

N = 10000
E = 320000
D = 128
H = 256
EPS = 1e-07
BN_EPS = 1e-05

NC = 2
NS = 16
CH = 120
NB = 3
NI = 6
NCHUNK = 168
NGRP = NCHUNK // NI
EPT = NCHUNK * CH
EPAD = EPT * NS
NPAD = 10112
RPT = NPAD // NS


def _prep_body(x_ref, t_ref, tab_ref):
    t = t_ref[0, 0]
    y = jnp.maximum(x_ref[...], 0.0) + EPS
    ey = jnp.exp(t * y)
    tab_ref[...] = jnp.zeros((2 * NPAD, D), jnp.float32)
    tab_ref[pl.ds(0, N), :] = ey
    tab_ref[pl.ds(NPAD, N), :] = y * ey


def _prep(x, t):
    return pl.pallas_call(
        _prep_body,
        out_shape=jax.ShapeDtypeStruct((2 * NPAD, D), jnp.float32),
    )(x, t.reshape(1, 1))


def _sc_body(tab_hbm, sd_hbm, zeros_hbm, out_hbm,
             acc, idxb, rows,
             i0, i1, g0, g1, g2, s0, s1, s2):
    isems = (i0, i1)
    gsems = (g0, g1, g2)
    ssems = (s0, s1, s2)
    c = lax.axis_index("c")
    s = lax.axis_index("s")

    pltpu.sync_copy(zeros_hbm, acc.at[pl.ds(s * RPT, RPT)])
    plsc.subcore_barrier()

    def issue_idx(j, slot):
        pltpu.async_copy(sd_hbm.at[c, s, j], idxb.at[slot], isems[slot])

    def wait_idx(slot):
        pltpu.make_async_copy(sd_hbm.at[0, 0, 0], idxb.at[slot],
                              isems[slot]).wait()

    def issue_gather(slot, u, b):
        pltpu.async_copy(tab_hbm.at[idxb.at[slot, u, 0]], rows.at[b],
                         gsems[b])

    def wait_gather(b):
        pltpu.make_async_copy(tab_hbm.at[idxb.at[0, 0, 0]], rows.at[b],
                              gsems[b]).wait()

    def issue_scatter(slot, u, b):
        pltpu.async_copy(rows.at[b], acc.at[idxb.at[slot, u, 1]], ssems[b],
                         add=True)

    def wait_scatter(b):
        pltpu.make_async_copy(rows.at[b], acc.at[idxb.at[0, 0, 1]],
                              ssems[b]).wait()

    issue_idx(0, 0)
    wait_idx(0)
    issue_gather(0, 0, 0)

    @pl.loop(0, NGRP // 2)
    def _grp2(gg):
        for gpar in range(2):
            g = gg * 2 + gpar
            for u in range(NI):
                k = g * NI + u
                b = u % NB

                wait_gather(b)
                issue_scatter(gpar, u, b)

                @pl.when(k >= 2)
                def _():
                    wait_scatter((b + 1) % NB)

                if u == 1:
                    @pl.when(g + 1 < NGRP)
                    def _():
                        issue_idx(g + 1, 1 - gpar)

                @pl.when(k + 1 < NCHUNK)
                def _():
                    if u == NI - 1:
                        wait_idx(1 - gpar)
                        issue_gather(1 - gpar, 0, (b + 1) % NB)
                    else:
                        issue_gather(gpar, u + 1, (b + 1) % NB)

    wait_scatter((NCHUNK - 2) % NB)
    wait_scatter((NCHUNK - 1) % NB)

    plsc.subcore_barrier()
    pltpu.sync_copy(acc.at[pl.ds(s * RPT, RPT)],
                    out_hbm.at[pl.ds(c * NPAD + s * RPT, RPT)])


_sc_edge = pl.kernel(
    _sc_body,
    out_type=jax.ShapeDtypeStruct((2 * NPAD, D), jnp.float32),
    mesh=plsc.VectorSubcoreMesh(core_axis_name="c", subcore_axis_name="s"),
    scratch_types=[
        pltpu.VMEM_SHARED((NPAD, D), jnp.float32),
        pltpu.VMEM((2, NI, 2, CH), jnp.int32),
        pltpu.VMEM((NB, CH, D), jnp.float32),
    ] + [pltpu.SemaphoreType.DMA] * 8,
)


TILE = 1000
GRID1 = N // TILE


def _mlp1_body(x_ref, den_ref, num_ref, w1_ref, b1_ref,
               h1_ref, s1_ref, s2_ref):
    i = pl.program_id(0)
    agg = num_ref[0] / (den_ref[0] + 1e-16)
    h = x_ref[...] + agg
    h1 = jnp.dot(h, w1_ref[...], preferred_element_type=jnp.float32)
    h1 = h1 + b1_ref[...]
    h1_ref[...] = h1
    ps1 = jnp.sum(h1, axis=0, keepdims=True)
    ps2 = jnp.sum(h1 * h1, axis=0, keepdims=True)

    @pl.when(i == 0)
    def _():
        s1_ref[...] = ps1
        s2_ref[...] = ps2

    @pl.when(i > 0)
    def _():
        s1_ref[...] += ps1
        s2_ref[...] += ps2


def _mlp1(x, sums, W1, b1):
    return pl.pallas_call(
        _mlp1_body,
        grid=(GRID1,),
        in_specs=[
            pl.BlockSpec((TILE, D), lambda i: (i, 0)),
            pl.BlockSpec((1, TILE, D), lambda i: (0, i, 0)),
            pl.BlockSpec((1, TILE, D), lambda i: (1, i, 0)),
            pl.BlockSpec((D, H), lambda i: (0, 0)),
            pl.BlockSpec((1, H), lambda i: (0, 0)),
        ],
        out_specs=[
            pl.BlockSpec((TILE, H), lambda i: (i, 0)),
            pl.BlockSpec((1, H), lambda i: (0, 0)),
            pl.BlockSpec((1, H), lambda i: (0, 0)),
        ],
        out_shape=[
            jax.ShapeDtypeStruct((N, H), jnp.float32),
            jax.ShapeDtypeStruct((1, H), jnp.float32),
            jax.ShapeDtypeStruct((1, H), jnp.float32),
        ],
    )(x, sums, sums, W1, b1.reshape(1, H))


def _mlp2_body(h1_ref, s1_ref, s2_ref, gamma_ref, beta_ref, w2_ref, b2_ref,
               out_ref):
    mean = s1_ref[...] / N
    var = s2_ref[...] / N - mean * mean
    scale = gamma_ref[...] * lax.rsqrt(var + BN_EPS)
    shift = beta_ref[...] - mean * scale
    h1 = h1_ref[...] * scale + shift
    h1 = jnp.maximum(h1, 0.0)
    out = jnp.dot(h1, w2_ref[...], preferred_element_type=jnp.float32)
    out_ref[...] = out + b2_ref[...]


def _mlp2(h1, s1, s2, gamma, beta, W2, b2):
    return pl.pallas_call(
        _mlp2_body,
        grid=(GRID1,),
        in_specs=[
            pl.BlockSpec((TILE, H), lambda i: (i, 0)),
            pl.BlockSpec((1, H), lambda i: (0, 0)),
            pl.BlockSpec((1, H), lambda i: (0, 0)),
            pl.BlockSpec((1, H), lambda i: (0, 0)),
            pl.BlockSpec((1, H), lambda i: (0, 0)),
            pl.BlockSpec((H, D), lambda i: (0, 0)),
            pl.BlockSpec((1, D), lambda i: (0, 0)),
        ],
        out_specs=pl.BlockSpec((TILE, D), lambda i: (i, 0)),
        out_shape=jax.ShapeDtypeStruct((N, D), jnp.float32),
    )(h1, s1, s2, gamma.reshape(1, H), beta.reshape(1, H), W2,
      b2.reshape(1, D))


def kernel(x, edge_index, t, W1, b1, gamma, beta, W2, b2):
    src = edge_index[0]
    dst = edge_index[1]
    pad = EPAD - E
    src_pad = jnp.concatenate([src, jnp.full((pad,), N, jnp.int32)])
    dst_pad = jnp.concatenate([dst, jnp.full((pad,), N, jnp.int32)])
    src4 = jnp.stack([src_pad, src_pad + NPAD]).reshape(2, NS, NCHUNK, 1, CH)
    dst4 = jnp.broadcast_to(dst_pad.reshape(1, NS, NCHUNK, 1, CH),
                            (2, NS, NCHUNK, 1, CH))
    sd = jnp.concatenate([src4, dst4], axis=3)
    sd = sd.reshape(2, NS, NGRP, NI, 2, CH)
    zeros = jnp.zeros((RPT, D), jnp.float32)

    tab = _prep(x, t)
    sums = _sc_edge(tab, sd, zeros)
    sums3 = sums.reshape(2, NPAD, D)
    h1, s1, s2 = _mlp1(x, sums3, W1, b1)
    return _mlp2(h1, s1, s2, gamma, beta, W2, b2)


# speedup vs baseline: 1.1345x; 1.1345x over previous
import functools

import jax
import jax.numpy as jnp
from jax import lax
from jax.experimental import pallas as pl
from jax.experimental.pallas import tpu as pltpu
from jax.experimental.pallas import tpu_sc as plsc

N = 10000
E = 320000
D = 128
H = 256
EPS = 1e-07
BN_EPS = 1e-05

NC = 2
NS = 16
CH = 120
NB = 3
NI = 6
NCHUNK = 168
NGRP = NCHUNK // NI
EPT = NCHUNK * CH
EPAD = EPT * NS
NPAD = 10112
RPT = NPAD // NS


def _prep_body(x_ref, t_ref, tab_ref):
    t = t_ref[0, 0]
    y = jnp.maximum(x_ref[...], 0.0) + EPS
    ey = jnp.exp(t * y)
    tab_ref[...] = jnp.zeros((2 * NPAD, D), jnp.float32)
    tab_ref[pl.ds(0, N), :] = ey
    tab_ref[pl.ds(NPAD, N), :] = y * ey


def _prep(x, t):
    return pl.pallas_call(
        _prep_body,
        out_shape=jax.ShapeDtypeStruct((2 * NPAD, D), jnp.float32),
    )(x, t.reshape(1, 1))


def _sc_body(tab_hbm, sd_hbm, zeros_hbm, out_hbm,
             acc, idxb, rows,
             i0, i1, g0, g1, g2, s0, s1, s2):
    isems = (i0, i1)
    gsems = (g0, g1, g2)
    ssems = (s0, s1, s2)
    c = lax.axis_index("c")
    s = lax.axis_index("s")

    pltpu.sync_copy(zeros_hbm, acc.at[pl.ds(s * RPT, RPT)])
    plsc.subcore_barrier()

    def issue_idx(j, slot):
        pltpu.async_copy(sd_hbm.at[c, s, j], idxb.at[slot], isems[slot])

    def wait_idx(slot):
        pltpu.make_async_copy(sd_hbm.at[0, 0, 0], idxb.at[slot],
                              isems[slot]).wait()

    def issue_gather(slot, u, b):
        pltpu.async_copy(tab_hbm.at[idxb.at[slot, u, 0]], rows.at[b],
                         gsems[b])

    def wait_gather(b):
        pltpu.make_async_copy(tab_hbm.at[idxb.at[0, 0, 0]], rows.at[b],
                              gsems[b]).wait()

    def issue_scatter(slot, u, b):
        pltpu.async_copy(rows.at[b], acc.at[idxb.at[slot, u, 1]], ssems[b],
                         add=True)

    def wait_scatter(b):
        pltpu.make_async_copy(rows.at[b], acc.at[idxb.at[0, 0, 1]],
                              ssems[b]).wait()

    issue_idx(0, 0)
    wait_idx(0)
    issue_gather(0, 0, 0)

    @pl.loop(0, NGRP // 2)
    def _grp2(gg):
        for gpar in range(2):
            g = gg * 2 + gpar
            for u in range(NI):
                k = g * NI + u
                b = u % NB

                @pl.when(k >= 2)
                def _():
                    wait_scatter((b + 1) % NB)

                @pl.when(k + 1 < NCHUNK)
                def _():
                    if u == NI - 1:
                        wait_idx(1 - gpar)
                        issue_gather(1 - gpar, 0, (b + 1) % NB)
                    else:
                        issue_gather(gpar, u + 1, (b + 1) % NB)

                if u == 1:
                    @pl.when(g + 1 < NGRP)
                    def _():
                        issue_idx(g + 1, 1 - gpar)

                wait_gather(b)
                issue_scatter(gpar, u, b)

    wait_scatter((NCHUNK - 2) % NB)
    wait_scatter((NCHUNK - 1) % NB)

    plsc.subcore_barrier()
    pltpu.sync_copy(acc.at[pl.ds(s * RPT, RPT)],
                    out_hbm.at[pl.ds(c * NPAD + s * RPT, RPT)])


_sc_edge = pl.kernel(
    _sc_body,
    out_type=jax.ShapeDtypeStruct((2 * NPAD, D), jnp.float32),
    mesh=plsc.VectorSubcoreMesh(core_axis_name="c", subcore_axis_name="s"),
    scratch_types=[
        pltpu.VMEM_SHARED((NPAD, D), jnp.float32),
        pltpu.VMEM((2, NI, 2, CH), jnp.int32),
        pltpu.VMEM((NB, CH, D), jnp.float32),
    ] + [pltpu.SemaphoreType.DMA] * 8,
)


TILE = 1000
GRID1 = N // TILE


def _mlp1_body(x_ref, den_ref, num_ref, w1_ref, b1_ref,
               h1_ref, s1_ref, s2_ref):
    i = pl.program_id(0)
    agg = num_ref[0] / (den_ref[0] + 1e-16)
    h = x_ref[...] + agg
    h1 = jnp.dot(h, w1_ref[...], preferred_element_type=jnp.float32)
    h1 = h1 + b1_ref[...]
    h1_ref[...] = h1
    ps1 = jnp.sum(h1, axis=0, keepdims=True)
    ps2 = jnp.sum(h1 * h1, axis=0, keepdims=True)

    @pl.when(i == 0)
    def _():
        s1_ref[...] = ps1
        s2_ref[...] = ps2

    @pl.when(i > 0)
    def _():
        s1_ref[...] += ps1
        s2_ref[...] += ps2


def _mlp1(x, sums, W1, b1):
    return pl.pallas_call(
        _mlp1_body,
        grid=(GRID1,),
        in_specs=[
            pl.BlockSpec((TILE, D), lambda i: (i, 0)),
            pl.BlockSpec((1, TILE, D), lambda i: (0, i, 0)),
            pl.BlockSpec((1, TILE, D), lambda i: (1, i, 0)),
            pl.BlockSpec((D, H), lambda i: (0, 0)),
            pl.BlockSpec((1, H), lambda i: (0, 0)),
        ],
        out_specs=[
            pl.BlockSpec((TILE, H), lambda i: (i, 0)),
            pl.BlockSpec((1, H), lambda i: (0, 0)),
            pl.BlockSpec((1, H), lambda i: (0, 0)),
        ],
        out_shape=[
            jax.ShapeDtypeStruct((N, H), jnp.float32),
            jax.ShapeDtypeStruct((1, H), jnp.float32),
            jax.ShapeDtypeStruct((1, H), jnp.float32),
        ],
    )(x, sums, sums, W1, b1.reshape(1, H))


def _mlp2_body(h1_ref, s1_ref, s2_ref, gamma_ref, beta_ref, w2_ref, b2_ref,
               out_ref):
    mean = s1_ref[...] / N
    var = s2_ref[...] / N - mean * mean
    scale = gamma_ref[...] * lax.rsqrt(var + BN_EPS)
    shift = beta_ref[...] - mean * scale
    h1 = h1_ref[...] * scale + shift
    h1 = jnp.maximum(h1, 0.0)
    out = jnp.dot(h1, w2_ref[...], preferred_element_type=jnp.float32)
    out_ref[...] = out + b2_ref[...]


def _mlp2(h1, s1, s2, gamma, beta, W2, b2):
    return pl.pallas_call(
        _mlp2_body,
        grid=(GRID1,),
        in_specs=[
            pl.BlockSpec((TILE, H), lambda i: (i, 0)),
            pl.BlockSpec((1, H), lambda i: (0, 0)),
            pl.BlockSpec((1, H), lambda i: (0, 0)),
            pl.BlockSpec((1, H), lambda i: (0, 0)),
            pl.BlockSpec((1, H), lambda i: (0, 0)),
            pl.BlockSpec((H, D), lambda i: (0, 0)),
            pl.BlockSpec((1, D), lambda i: (0, 0)),
        ],
        out_specs=pl.BlockSpec((TILE, D), lambda i: (i, 0)),
        out_shape=jax.ShapeDtypeStruct((N, D), jnp.float32),
    )(h1, s1, s2, gamma.reshape(1, H), beta.reshape(1, H), W2,
      b2.reshape(1, D))


def kernel(x, edge_index, t, W1, b1, gamma, beta, W2, b2):
    src = edge_index[0]
    dst = edge_index[1]
    pad = EPAD - E
    src_pad = jnp.concatenate([src, jnp.full((pad,), N, jnp.int32)])
    dst_pad = jnp.concatenate([dst, jnp.full((pad,), N, jnp.int32)])
    src4 = jnp.stack([src_pad, src_pad + NPAD]).reshape(2, NS, NCHUNK, 1, CH)
    dst4 = jnp.broadcast_to(dst_pad.reshape(1, NS, NCHUNK, 1, CH),
                            (2, NS, NCHUNK, 1, CH))
    sd = jnp.concatenate([src4, dst4], axis=3)
    sd = sd.reshape(2, NS, NGRP, NI, 2, CH)
    zeros = jnp.zeros((RPT, D), jnp.float32)

    tab = _prep(x, t)
    sums = _sc_edge(tab, sd, zeros)
    sums3 = sums.reshape(2, NPAD, D)
    h1, s1, s2 = _mlp1(x, sums3, W1, b1)
    return _mlp2(h1, s1, s2, gamma, beta, W2, b2)


# speedup vs baseline: 1.3417x; 1.1826x over previous
import functools

import jax
import jax.numpy as jnp
from jax import lax
from jax.experimental import pallas as pl
from jax.experimental.pallas import tpu as pltpu
from jax.experimental.pallas import tpu_sc as plsc

N = 10000
E = 320000
D = 128
H = 256
EPS = 1e-07
BN_EPS = 1e-05

NC = 2
NS = 16
CH = 88
NB = 4
NI = 6
NCHUNK = 228
NGRP = NCHUNK // NI
EPT = NCHUNK * CH
EPAD = EPT * NS
NPAD = 10112
RPT = NPAD // NS


def _prep_body(x_ref, t_ref, tab_ref):
    t = t_ref[0, 0]
    y = jnp.maximum(x_ref[...], 0.0) + EPS
    ey = jnp.exp(t * y)
    tab_ref[...] = jnp.zeros((2 * NPAD, D), jnp.float32)
    tab_ref[pl.ds(0, N), :] = ey
    tab_ref[pl.ds(NPAD, N), :] = y * ey


def _prep(x, t):
    return pl.pallas_call(
        _prep_body,
        out_shape=jax.ShapeDtypeStruct((2 * NPAD, D), jnp.float32),
    )(x, t.reshape(1, 1))


def _sc_body(tab_hbm, sd_hbm, zeros_hbm, out_hbm,
             acc, idxb, rows,
             i0, i1, g0, g1, g2, g3, s0, s1, s2, s3):
    isems = (i0, i1)
    gsems = (g0, g1, g2, g3)
    ssems = (s0, s1, s2, s3)
    c = lax.axis_index("c")
    s = lax.axis_index("s")

    pltpu.sync_copy(zeros_hbm, acc.at[pl.ds(s * RPT, RPT)])
    plsc.subcore_barrier()

    def issue_idx(j, slot):
        pltpu.async_copy(sd_hbm.at[c, s, j], idxb.at[slot], isems[slot])

    def wait_idx(slot):
        pltpu.make_async_copy(sd_hbm.at[0, 0, 0], idxb.at[slot],
                              isems[slot]).wait()

    def issue_gather(slot, u, b):
        pltpu.async_copy(tab_hbm.at[idxb.at[slot, u, 0]], rows.at[b],
                         gsems[b])

    def wait_gather(b):
        pltpu.make_async_copy(tab_hbm.at[idxb.at[0, 0, 0]], rows.at[b],
                              gsems[b]).wait()

    def issue_scatter(slot, u, b):
        pltpu.async_copy(rows.at[b], acc.at[idxb.at[slot, u, 1]], ssems[b],
                         add=True)

    def wait_scatter(b):
        pltpu.make_async_copy(rows.at[b], acc.at[idxb.at[0, 0, 1]],
                              ssems[b]).wait()

    issue_idx(0, 0)
    wait_idx(0)
    issue_gather(0, 0, 0)
    issue_gather(0, 1, 1)

    @pl.loop(0, NGRP // 2)
    def _grp2(gg):
        for gpar in range(2):
            g = gg * 2 + gpar
            for u in range(NI):
                k = g * NI + u
                b = (u + 2 * gpar) % NB

                @pl.when(k >= 2)
                def _():
                    wait_scatter((b + 2) % NB)

                if u == 1:
                    @pl.when(g + 1 < NGRP)
                    def _():
                        issue_idx(g + 1, 1 - gpar)

                if u == 4:
                    @pl.when(k + 2 < NCHUNK)
                    def _():
                        wait_idx(1 - gpar)

                @pl.when(k + 2 < NCHUNK)
                def _():
                    if u < 4:
                        issue_gather(gpar, u + 2, (b + 2) % NB)
                    else:
                        issue_gather(1 - gpar, u - 4, (b + 2) % NB)

                wait_gather(b)
                issue_scatter(gpar, u, b)

    wait_scatter(2)
    wait_scatter(3)

    plsc.subcore_barrier()
    pltpu.sync_copy(acc.at[pl.ds(s * RPT, RPT)],
                    out_hbm.at[pl.ds(c * NPAD + s * RPT, RPT)])


_sc_edge = pl.kernel(
    _sc_body,
    out_type=jax.ShapeDtypeStruct((2 * NPAD, D), jnp.float32),
    mesh=plsc.VectorSubcoreMesh(core_axis_name="c", subcore_axis_name="s"),
    scratch_types=[
        pltpu.VMEM_SHARED((NPAD, D), jnp.float32),
        pltpu.VMEM((2, NI, 2, CH), jnp.int32),
        pltpu.VMEM((NB, CH, D), jnp.float32),
    ] + [pltpu.SemaphoreType.DMA] * 10,
)


TILE = 1000
GRID1 = N // TILE


def _mlp1_body(x_ref, den_ref, num_ref, w1_ref, b1_ref,
               h1_ref, s1_ref, s2_ref):
    i = pl.program_id(0)
    agg = num_ref[0] / (den_ref[0] + 1e-16)
    h = x_ref[...] + agg
    h1 = jnp.dot(h, w1_ref[...], preferred_element_type=jnp.float32)
    h1 = h1 + b1_ref[...]
    h1_ref[...] = h1
    ps1 = jnp.sum(h1, axis=0, keepdims=True)
    ps2 = jnp.sum(h1 * h1, axis=0, keepdims=True)

    @pl.when(i == 0)
    def _():
        s1_ref[...] = ps1
        s2_ref[...] = ps2

    @pl.when(i > 0)
    def _():
        s1_ref[...] += ps1
        s2_ref[...] += ps2


def _mlp1(x, sums, W1, b1):
    return pl.pallas_call(
        _mlp1_body,
        grid=(GRID1,),
        in_specs=[
            pl.BlockSpec((TILE, D), lambda i: (i, 0)),
            pl.BlockSpec((1, TILE, D), lambda i: (0, i, 0)),
            pl.BlockSpec((1, TILE, D), lambda i: (1, i, 0)),
            pl.BlockSpec((D, H), lambda i: (0, 0)),
            pl.BlockSpec((1, H), lambda i: (0, 0)),
        ],
        out_specs=[
            pl.BlockSpec((TILE, H), lambda i: (i, 0)),
            pl.BlockSpec((1, H), lambda i: (0, 0)),
            pl.BlockSpec((1, H), lambda i: (0, 0)),
        ],
        out_shape=[
            jax.ShapeDtypeStruct((N, H), jnp.float32),
            jax.ShapeDtypeStruct((1, H), jnp.float32),
            jax.ShapeDtypeStruct((1, H), jnp.float32),
        ],
    )(x, sums, sums, W1, b1.reshape(1, H))


def _mlp2_body(h1_ref, s1_ref, s2_ref, gamma_ref, beta_ref, w2_ref, b2_ref,
               out_ref):
    mean = s1_ref[...] / N
    var = s2_ref[...] / N - mean * mean
    scale = gamma_ref[...] * lax.rsqrt(var + BN_EPS)
    shift = beta_ref[...] - mean * scale
    h1 = h1_ref[...] * scale + shift
    h1 = jnp.maximum(h1, 0.0)
    out = jnp.dot(h1, w2_ref[...], preferred_element_type=jnp.float32)
    out_ref[...] = out + b2_ref[...]


def _mlp2(h1, s1, s2, gamma, beta, W2, b2):
    return pl.pallas_call(
        _mlp2_body,
        grid=(GRID1,),
        in_specs=[
            pl.BlockSpec((TILE, H), lambda i: (i, 0)),
            pl.BlockSpec((1, H), lambda i: (0, 0)),
            pl.BlockSpec((1, H), lambda i: (0, 0)),
            pl.BlockSpec((1, H), lambda i: (0, 0)),
            pl.BlockSpec((1, H), lambda i: (0, 0)),
            pl.BlockSpec((H, D), lambda i: (0, 0)),
            pl.BlockSpec((1, D), lambda i: (0, 0)),
        ],
        out_specs=pl.BlockSpec((TILE, D), lambda i: (i, 0)),
        out_shape=jax.ShapeDtypeStruct((N, D), jnp.float32),
    )(h1, s1, s2, gamma.reshape(1, H), beta.reshape(1, H), W2,
      b2.reshape(1, D))


def kernel(x, edge_index, t, W1, b1, gamma, beta, W2, b2):
    src = edge_index[0]
    dst = edge_index[1]
    pad = EPAD - E
    src_pad = jnp.concatenate([src, jnp.full((pad,), N, jnp.int32)])
    dst_pad = jnp.concatenate([dst, jnp.full((pad,), N, jnp.int32)])
    src4 = jnp.stack([src_pad, src_pad + NPAD]).reshape(2, NS, NCHUNK, 1, CH)
    dst4 = jnp.broadcast_to(dst_pad.reshape(1, NS, NCHUNK, 1, CH),
                            (2, NS, NCHUNK, 1, CH))
    sd = jnp.concatenate([src4, dst4], axis=3)
    sd = sd.reshape(2, NS, NGRP, NI, 2, CH)
    zeros = jnp.zeros((RPT, D), jnp.float32)

    tab = _prep(x, t)
    sums = _sc_edge(tab, sd, zeros)
    sums3 = sums.reshape(2, NPAD, D)
    h1, s1, s2 = _mlp1(x, sums3, W1, b1)
    return _mlp2(h1, s1, s2, gamma, beta, W2, b2)


# speedup vs baseline: 1.3855x; 1.0326x over previous
import functools

import jax
import jax.numpy as jnp
from jax import lax
from jax.experimental import pallas as pl
from jax.experimental.pallas import tpu as pltpu
from jax.experimental.pallas import tpu_sc as plsc

N = 10000
E = 320000
D = 128
H = 256
EPS = 1e-07
BN_EPS = 1e-05

NC = 2
NS = 16
CH = 88
NB = 4
NI = 6
NCHUNK = 228
NGRP = NCHUNK // NI
EPT = NCHUNK * CH
EPAD = EPT * NS
NPAD = 10112
RPT = NPAD // NS


def _prep_body(x_ref, t_ref, tab_ref):
    t = t_ref[0, 0]
    y = jnp.maximum(x_ref[...], 0.0) + EPS
    ey = jnp.exp(t * y)
    tab_ref[...] = jnp.zeros((2 * NPAD, D), jnp.float32)
    tab_ref[pl.ds(0, N), :] = ey
    tab_ref[pl.ds(NPAD, N), :] = y * ey


def _prep(x, t):
    return pl.pallas_call(
        _prep_body,
        out_shape=jax.ShapeDtypeStruct((2 * NPAD, D), jnp.float32),
    )(x, t.reshape(1, 1))


def _sc_body(tab_hbm, sd_hbm, zeros_hbm, out_hbm,
             acc, idxb, rows,
             i0, i1, g0, g1, g2, g3, s0, s1, s2, s3):
    isems = (i0, i1)
    gsems = (g0, g1, g2, g3)
    ssems = (s0, s1, s2, s3)
    c = lax.axis_index("c")
    s = lax.axis_index("s")

    pltpu.sync_copy(zeros_hbm, acc.at[pl.ds(s * RPT, RPT)])
    plsc.subcore_barrier()

    def issue_idx(j, slot):
        pltpu.async_copy(sd_hbm.at[c, s, j], idxb.at[slot], isems[slot])

    def wait_idx(slot):
        pltpu.make_async_copy(sd_hbm.at[0, 0, 0], idxb.at[slot],
                              isems[slot]).wait()

    def issue_gather(slot, u, b):
        pltpu.async_copy(tab_hbm.at[idxb.at[slot, u, 0]], rows.at[b],
                         gsems[b])

    def wait_gather(b):
        pltpu.make_async_copy(tab_hbm.at[idxb.at[0, 0, 0]], rows.at[b],
                              gsems[b]).wait()

    def issue_scatter(slot, u, b):
        pltpu.async_copy(rows.at[b], acc.at[idxb.at[slot, u, 1]], ssems[b],
                         add=True)

    def wait_scatter(b):
        pltpu.make_async_copy(rows.at[b], acc.at[idxb.at[0, 0, 1]],
                              ssems[b]).wait()

    issue_idx(0, 0)
    wait_idx(0)
    issue_gather(0, 0, 0)
    issue_gather(0, 1, 1)
    issue_gather(0, 2, 2)

    @pl.loop(0, NGRP // 2)
    def _grp2(gg):
        for gpar in range(2):
            g = gg * 2 + gpar
            for u in range(NI):
                k = g * NI + u
                b = (u + 2 * gpar) % NB

                @pl.when(k >= 1)
                def _():
                    wait_scatter((b + 3) % NB)

                if u == 1:
                    @pl.when(g + 1 < NGRP)
                    def _():
                        issue_idx(g + 1, 1 - gpar)

                if u == 3:
                    @pl.when(k + 3 < NCHUNK)
                    def _():
                        wait_idx(1 - gpar)

                @pl.when(k + 3 < NCHUNK)
                def _():
                    if u < 3:
                        issue_gather(gpar, u + 3, (b + 3) % NB)
                    else:
                        issue_gather(1 - gpar, u - 3, (b + 3) % NB)

                wait_gather(b)
                issue_scatter(gpar, u, b)

    wait_scatter(3)

    plsc.subcore_barrier()
    pltpu.sync_copy(acc.at[pl.ds(s * RPT, RPT)],
                    out_hbm.at[pl.ds(c * NPAD + s * RPT, RPT)])


_sc_edge = pl.kernel(
    _sc_body,
    out_type=jax.ShapeDtypeStruct((2 * NPAD, D), jnp.float32),
    mesh=plsc.VectorSubcoreMesh(core_axis_name="c", subcore_axis_name="s"),
    scratch_types=[
        pltpu.VMEM_SHARED((NPAD, D), jnp.float32),
        pltpu.VMEM((2, NI, 2, CH), jnp.int32),
        pltpu.VMEM((NB, CH, D), jnp.float32),
    ] + [pltpu.SemaphoreType.DMA] * 10,
)


TILE = 1000
GRID1 = N // TILE


def _mlp1_body(x_ref, den_ref, num_ref, w1_ref, b1_ref,
               h1_ref, s1_ref, s2_ref):
    i = pl.program_id(0)
    agg = num_ref[0] / (den_ref[0] + 1e-16)
    h = x_ref[...] + agg
    h1 = jnp.dot(h, w1_ref[...], preferred_element_type=jnp.float32)
    h1 = h1 + b1_ref[...]
    h1_ref[...] = h1
    ps1 = jnp.sum(h1, axis=0, keepdims=True)
    ps2 = jnp.sum(h1 * h1, axis=0, keepdims=True)

    @pl.when(i == 0)
    def _():
        s1_ref[...] = ps1
        s2_ref[...] = ps2

    @pl.when(i > 0)
    def _():
        s1_ref[...] += ps1
        s2_ref[...] += ps2


def _mlp1(x, sums, W1, b1):
    return pl.pallas_call(
        _mlp1_body,
        grid=(GRID1,),
        in_specs=[
            pl.BlockSpec((TILE, D), lambda i: (i, 0)),
            pl.BlockSpec((1, TILE, D), lambda i: (0, i, 0)),
            pl.BlockSpec((1, TILE, D), lambda i: (1, i, 0)),
            pl.BlockSpec((D, H), lambda i: (0, 0)),
            pl.BlockSpec((1, H), lambda i: (0, 0)),
        ],
        out_specs=[
            pl.BlockSpec((TILE, H), lambda i: (i, 0)),
            pl.BlockSpec((1, H), lambda i: (0, 0)),
            pl.BlockSpec((1, H), lambda i: (0, 0)),
        ],
        out_shape=[
            jax.ShapeDtypeStruct((N, H), jnp.float32),
            jax.ShapeDtypeStruct((1, H), jnp.float32),
            jax.ShapeDtypeStruct((1, H), jnp.float32),
        ],
    )(x, sums, sums, W1, b1.reshape(1, H))


def _mlp2_body(h1_ref, s1_ref, s2_ref, gamma_ref, beta_ref, w2_ref, b2_ref,
               out_ref):
    mean = s1_ref[...] / N
    var = s2_ref[...] / N - mean * mean
    scale = gamma_ref[...] * lax.rsqrt(var + BN_EPS)
    shift = beta_ref[...] - mean * scale
    h1 = h1_ref[...] * scale + shift
    h1 = jnp.maximum(h1, 0.0)
    out = jnp.dot(h1, w2_ref[...], preferred_element_type=jnp.float32)
    out_ref[...] = out + b2_ref[...]


def _mlp2(h1, s1, s2, gamma, beta, W2, b2):
    return pl.pallas_call(
        _mlp2_body,
        grid=(GRID1,),
        in_specs=[
            pl.BlockSpec((TILE, H), lambda i: (i, 0)),
            pl.BlockSpec((1, H), lambda i: (0, 0)),
            pl.BlockSpec((1, H), lambda i: (0, 0)),
            pl.BlockSpec((1, H), lambda i: (0, 0)),
            pl.BlockSpec((1, H), lambda i: (0, 0)),
            pl.BlockSpec((H, D), lambda i: (0, 0)),
            pl.BlockSpec((1, D), lambda i: (0, 0)),
        ],
        out_specs=pl.BlockSpec((TILE, D), lambda i: (i, 0)),
        out_shape=jax.ShapeDtypeStruct((N, D), jnp.float32),
    )(h1, s1, s2, gamma.reshape(1, H), beta.reshape(1, H), W2,
      b2.reshape(1, D))


def kernel(x, edge_index, t, W1, b1, gamma, beta, W2, b2):
    src = edge_index[0]
    dst = edge_index[1]
    pad = EPAD - E
    src_pad = jnp.concatenate([src, jnp.full((pad,), N, jnp.int32)])
    dst_pad = jnp.concatenate([dst, jnp.full((pad,), N, jnp.int32)])
    src4 = jnp.stack([src_pad, src_pad + NPAD]).reshape(2, NS, NCHUNK, 1, CH)
    dst4 = jnp.broadcast_to(dst_pad.reshape(1, NS, NCHUNK, 1, CH),
                            (2, NS, NCHUNK, 1, CH))
    sd = jnp.concatenate([src4, dst4], axis=3)
    sd = sd.reshape(2, NS, NGRP, NI, 2, CH)
    zeros = jnp.zeros((RPT, D), jnp.float32)

    tab = _prep(x, t)
    sums = _sc_edge(tab, sd, zeros)
    sums3 = sums.reshape(2, NPAD, D)
    h1, s1, s2 = _mlp1(x, sums3, W1, b1)
    return _mlp2(h1, s1, s2, gamma, beta, W2, b2)


# speedup vs baseline: 1.4204x; 1.0252x over previous
import functools

import jax
import jax.numpy as jnp
from jax import lax
from jax.experimental import pallas as pl
from jax.experimental.pallas import tpu as pltpu
from jax.experimental.pallas import tpu_sc as plsc

N = 10000
E = 320000
D = 128
H = 256
EPS = 1e-07
BN_EPS = 1e-05

NC = 2
NS = 16
CH = 88
NB = 4
NI = 6
NCHUNK = 228
NGRP = NCHUNK // NI
EPT = NCHUNK * CH
EPAD = EPT * NS
NPAD = 10112
RPT = NPAD // NS


def _prep_body(x_ref, t_ref, tab_ref):
    t = t_ref[0, 0]
    y = jnp.maximum(x_ref[...], 0.0) + EPS
    ey = jnp.exp(t * y)
    tab_ref[...] = jnp.zeros((2 * NPAD, D), jnp.float32)
    tab_ref[pl.ds(0, N), :] = ey
    tab_ref[pl.ds(NPAD, N), :] = y * ey


def _prep(x, t):
    return pl.pallas_call(
        _prep_body,
        out_shape=jax.ShapeDtypeStruct((2 * NPAD, D), jnp.float32),
    )(x, t.reshape(1, 1))


def _sc_body(tab_hbm, sd_hbm, zeros_hbm, out_hbm,
             acc, idxb, rows,
             i0, i1, g0, g1, g2, g3, s0, s1, s2, s3):
    isems = (i0, i1)
    gsems = (g0, g1, g2, g3)
    ssems = (s0, s1, s2, s3)
    c = lax.axis_index("c")
    s = lax.axis_index("s")

    pltpu.sync_copy(zeros_hbm, acc.at[pl.ds(s * RPT, RPT)])
    plsc.subcore_barrier()

    def issue_idx(j, slot):
        pltpu.async_copy(sd_hbm.at[c, s, j], idxb.at[slot], isems[slot])

    def wait_idx(slot):
        pltpu.make_async_copy(sd_hbm.at[0, 0, 0], idxb.at[slot],
                              isems[slot]).wait()

    def issue_gather(slot, u, b):
        pltpu.async_copy(tab_hbm.at[idxb.at[slot, u, 0]], rows.at[b],
                         gsems[b])

    def wait_gather(b):
        pltpu.make_async_copy(tab_hbm.at[idxb.at[0, 0, 0]], rows.at[b],
                              gsems[b]).wait()

    def issue_scatter(slot, u, b):
        pltpu.async_copy(rows.at[b], acc.at[idxb.at[slot, u, 1]], ssems[b],
                         add=True)

    def wait_scatter(b):
        pltpu.make_async_copy(rows.at[b], acc.at[idxb.at[0, 0, 1]],
                              ssems[b]).wait()

    issue_idx(0, 0)
    wait_idx(0)
    issue_gather(0, 0, 0)
    issue_gather(0, 1, 1)
    issue_gather(0, 2, 2)

    @pl.loop(0, NGRP // 2)
    def _grp2(gg):
        for gpar in range(2):
            g = gg * 2 + gpar
            for u in range(NI):
                k = g * NI + u
                b = (u + 2 * gpar) % NB

                @pl.when(k >= 1)
                def _():
                    wait_scatter((b + 3) % NB)

                if u == 1:
                    @pl.when(g + 1 < NGRP)
                    def _():
                        issue_idx(g + 1, 1 - gpar)

                if u == 3:
                    @pl.when(k + 3 < NCHUNK)
                    def _():
                        wait_idx(1 - gpar)

                @pl.when(k + 3 < NCHUNK)
                def _():
                    if u < 3:
                        issue_gather(gpar, u + 3, (b + 3) % NB)
                    else:
                        issue_gather(1 - gpar, u - 3, (b + 3) % NB)

                wait_gather(b)
                issue_scatter(gpar, u, b)

    wait_scatter(3)

    plsc.subcore_barrier()
    pltpu.sync_copy(acc.at[pl.ds(s * RPT, RPT)],
                    out_hbm.at[pl.ds(c * NPAD + s * RPT, RPT)])


_sc_edge = pl.kernel(
    _sc_body,
    out_type=jax.ShapeDtypeStruct((2 * NPAD, D), jnp.float32),
    mesh=plsc.VectorSubcoreMesh(core_axis_name="c", subcore_axis_name="s"),
    scratch_types=[
        pltpu.VMEM_SHARED((NPAD, D), jnp.float32),
        pltpu.VMEM((2, NI, 2, CH), jnp.int32),
        pltpu.VMEM((NB, CH, D), jnp.float32),
    ] + [pltpu.SemaphoreType.DMA] * 10,
)


TILE = 1000
GRID1 = N // TILE


def _mlp_body(x_ref, den_ref, num_ref, w1_ref, b1_ref,
              gamma_ref, beta_ref, w2_ref, b2_ref,
              out_ref, h1_ref, s1_ref, s2_ref):
    i = pl.program_id(0)

    @pl.when(i < GRID1)
    def _():
        agg = num_ref[0] / (den_ref[0] + 1e-16)
        h = x_ref[...] + agg
        h1 = jnp.dot(h, w1_ref[...], preferred_element_type=jnp.float32)
        h1 = h1 + b1_ref[...]
        h1_ref[i, :, :] = h1
        ps1 = jnp.sum(h1, axis=0, keepdims=True)
        ps2 = jnp.sum(h1 * h1, axis=0, keepdims=True)

        @pl.when(i == 0)
        def _():
            s1_ref[...] = ps1
            s2_ref[...] = ps2

        @pl.when(i > 0)
        def _():
            s1_ref[...] += ps1
            s2_ref[...] += ps2

    @pl.when(i >= GRID1)
    def _():
        mean = s1_ref[...] / N
        var = s2_ref[...] / N - mean * mean
        scale = gamma_ref[...] * lax.rsqrt(var + BN_EPS)
        shift = beta_ref[...] - mean * scale
        h1 = h1_ref[i - GRID1, :, :] * scale + shift
        h1 = jnp.maximum(h1, 0.0)
        out = jnp.dot(h1, w2_ref[...], preferred_element_type=jnp.float32)
        out_ref[...] = out + b2_ref[...]


def _mlp(x, sums, W1, b1, gamma, beta, W2, b2):
    pin = jnp.minimum
    return pl.pallas_call(
        _mlp_body,
        grid=(2 * GRID1,),
        in_specs=[
            pl.BlockSpec((TILE, D), lambda i: (pin(i, GRID1 - 1), 0)),
            pl.BlockSpec((1, TILE, D), lambda i: (0, pin(i, GRID1 - 1), 0)),
            pl.BlockSpec((1, TILE, D), lambda i: (1, pin(i, GRID1 - 1), 0)),
            pl.BlockSpec((D, H), lambda i: (0, 0)),
            pl.BlockSpec((1, H), lambda i: (0, 0)),
            pl.BlockSpec((1, H), lambda i: (0, 0)),
            pl.BlockSpec((1, H), lambda i: (0, 0)),
            pl.BlockSpec((H, D), lambda i: (0, 0)),
            pl.BlockSpec((1, D), lambda i: (0, 0)),
        ],
        out_specs=pl.BlockSpec(
            (TILE, D), lambda i: (jnp.maximum(i - GRID1, 0), 0)),
        out_shape=jax.ShapeDtypeStruct((N, D), jnp.float32),
        scratch_shapes=[
            pltpu.VMEM((GRID1, TILE, H), jnp.float32),
            pltpu.VMEM((1, H), jnp.float32),
            pltpu.VMEM((1, H), jnp.float32),
        ],
    )(x, sums, sums, W1, b1.reshape(1, H), gamma.reshape(1, H),
      beta.reshape(1, H), W2, b2.reshape(1, D))


def kernel(x, edge_index, t, W1, b1, gamma, beta, W2, b2):
    src = edge_index[0]
    dst = edge_index[1]
    pad = EPAD - E
    src_pad = jnp.concatenate([src, jnp.full((pad,), N, jnp.int32)])
    dst_pad = jnp.concatenate([dst, jnp.full((pad,), N, jnp.int32)])
    src4 = jnp.stack([src_pad, src_pad + NPAD]).reshape(2, NS, NCHUNK, 1, CH)
    dst4 = jnp.broadcast_to(dst_pad.reshape(1, NS, NCHUNK, 1, CH),
                            (2, NS, NCHUNK, 1, CH))
    sd = jnp.concatenate([src4, dst4], axis=3)
    sd = sd.reshape(2, NS, NGRP, NI, 2, CH)
    zeros = jnp.zeros((RPT, D), jnp.float32)

    tab = _prep(x, t)
    sums = _sc_edge(tab, sd, zeros)
    sums3 = sums.reshape(2, NPAD, D)
    return _mlp(x, sums3, W1, b1, gamma, beta, W2, b2)


# speedup vs baseline: 1.4221x; 1.0012x over previous
import jax
import jax.numpy as jnp
from jax import lax
from jax.experimental import pallas as pl
from jax.experimental.pallas import tpu as pltpu
from jax.experimental.pallas import tpu_sc as plsc

N = 10000
E = 320000
D = 128
H = 256
EPS = 1e-07
BN_EPS = 1e-05

NC = 2
NS = 16
CH = 88
NB = 4
NI = 6
NCHUNK = 228
NGRP = NCHUNK // NI
EPT = NCHUNK * CH
EPAD = EPT * NS
NPAD = 10112
RPT = NPAD // NS


def _prep_body(x_ref, t_ref, tab_ref):
    t = t_ref[0, 0]
    y = jnp.maximum(x_ref[...], 0.0) + EPS
    ey = jnp.exp(t * y)
    tab_ref[...] = jnp.zeros((2 * NPAD, D), jnp.float32)
    tab_ref[pl.ds(0, N), :] = ey
    tab_ref[pl.ds(NPAD, N), :] = y * ey


def _prep(x, t):
    return pl.pallas_call(
        _prep_body,
        out_shape=jax.ShapeDtypeStruct((2 * NPAD, D), jnp.float32),
    )(x, t.reshape(1, 1))


def _sc_body(tab_hbm, sd_hbm, zeros_hbm, out_hbm,
             acc, idxb, rows,
             i0, i1, g0, g1, g2, g3, s0, s1, s2, s3):
    isems = (i0, i1)
    gsems = (g0, g1, g2, g3)
    ssems = (s0, s1, s2, s3)
    c = lax.axis_index("c")
    s = lax.axis_index("s")

    pltpu.sync_copy(zeros_hbm, acc.at[pl.ds(s * RPT, RPT)])
    plsc.subcore_barrier()

    def issue_idx(j, slot):
        pltpu.async_copy(sd_hbm.at[c, s, j], idxb.at[slot], isems[slot])

    def wait_idx(slot):
        pltpu.make_async_copy(sd_hbm.at[0, 0, 0], idxb.at[slot],
                              isems[slot]).wait()

    def issue_gather(slot, u, b):
        pltpu.async_copy(tab_hbm.at[idxb.at[slot, u, 0]], rows.at[b],
                         gsems[b])

    def wait_gather(b):
        pltpu.make_async_copy(tab_hbm.at[idxb.at[0, 0, 0]], rows.at[b],
                              gsems[b]).wait()

    def issue_scatter(slot, u, b):
        pltpu.async_copy(rows.at[b], acc.at[idxb.at[slot, u, 1]], ssems[b],
                         add=True)

    def wait_scatter(b):
        pltpu.make_async_copy(rows.at[b], acc.at[idxb.at[0, 0, 1]],
                              ssems[b]).wait()

    issue_idx(0, 0)
    wait_idx(0)
    issue_gather(0, 0, 0)
    issue_gather(0, 1, 1)
    issue_gather(0, 2, 2)

    @pl.loop(0, NGRP // 2)
    def _grp2(gg):
        for gpar in range(2):
            g = gg * 2 + gpar
            for u in range(NI):
                k = g * NI + u
                b = (u + 2 * gpar) % NB

                @pl.when(k >= 1)
                def _():
                    wait_scatter((b + 3) % NB)

                if u == 1:
                    @pl.when(g + 1 < NGRP)
                    def _():
                        issue_idx(g + 1, 1 - gpar)

                if u == 3:
                    @pl.when(k + 3 < NCHUNK)
                    def _():
                        wait_idx(1 - gpar)

                @pl.when(k + 3 < NCHUNK)
                def _():
                    if u < 3:
                        issue_gather(gpar, u + 3, (b + 3) % NB)
                    else:
                        issue_gather(1 - gpar, u - 3, (b + 3) % NB)

                wait_gather(b)
                issue_scatter(gpar, u, b)

    wait_scatter(3)

    plsc.subcore_barrier()
    pltpu.sync_copy(acc.at[pl.ds(s * RPT, RPT)],
                    out_hbm.at[pl.ds(c * NPAD + s * RPT, RPT)])


_sc_edge = pl.kernel(
    _sc_body,
    out_type=jax.ShapeDtypeStruct((2 * NPAD, D), jnp.float32),
    mesh=plsc.VectorSubcoreMesh(core_axis_name="c", subcore_axis_name="s"),
    scratch_types=[
        pltpu.VMEM_SHARED((NPAD, D), jnp.float32),
        pltpu.VMEM((2, NI, 2, CH), jnp.int32),
        pltpu.VMEM((NB, CH, D), jnp.float32),
    ] + [pltpu.SemaphoreType.DMA] * 10,
)


TILE = 1000
GRID1 = N // TILE


def _mlp_body(x_ref, den_ref, num_ref, w1_ref, b1_ref,
              gamma_ref, beta_ref, w2_ref, b2_ref,
              out_ref, h1_ref, s1_ref, s2_ref):
    i = pl.program_id(0)

    @pl.when(i < GRID1)
    def _():
        agg = num_ref[0] / (den_ref[0] + 1e-16)
        h = x_ref[...] + agg
        h1 = jnp.dot(h, w1_ref[...], preferred_element_type=jnp.float32)
        h1 = h1 + b1_ref[...]
        h1_ref[i, :, :] = h1
        ps1 = jnp.sum(h1, axis=0, keepdims=True)
        ps2 = jnp.sum(h1 * h1, axis=0, keepdims=True)

        @pl.when(i == 0)
        def _():
            s1_ref[...] = ps1
            s2_ref[...] = ps2

        @pl.when(i > 0)
        def _():
            s1_ref[...] += ps1
            s2_ref[...] += ps2

    @pl.when(i >= GRID1)
    def _():
        mean = s1_ref[...] / N
        var = s2_ref[...] / N - mean * mean
        scale = gamma_ref[...] * lax.rsqrt(var + BN_EPS)
        shift = beta_ref[...] - mean * scale
        h1 = h1_ref[i - GRID1, :, :] * scale + shift
        h1 = jnp.maximum(h1, 0.0)
        out = jnp.dot(h1, w2_ref[...], preferred_element_type=jnp.float32)
        out_ref[...] = out + b2_ref[...]


def _mlp(x, sums, W1, b1, gamma, beta, W2, b2):
    pin = jnp.minimum
    return pl.pallas_call(
        _mlp_body,
        grid=(2 * GRID1,),
        in_specs=[
            pl.BlockSpec((TILE, D), lambda i: (pin(i, GRID1 - 1), 0)),
            pl.BlockSpec((1, TILE, D), lambda i: (0, pin(i, GRID1 - 1), 0)),
            pl.BlockSpec((1, TILE, D), lambda i: (1, pin(i, GRID1 - 1), 0)),
            pl.BlockSpec((D, H), lambda i: (0, 0)),
            pl.BlockSpec((1, H), lambda i: (0, 0)),
            pl.BlockSpec((1, H), lambda i: (0, 0)),
            pl.BlockSpec((1, H), lambda i: (0, 0)),
            pl.BlockSpec((H, D), lambda i: (0, 0)),
            pl.BlockSpec((1, D), lambda i: (0, 0)),
        ],
        out_specs=pl.BlockSpec(
            (TILE, D), lambda i: (jnp.maximum(i - GRID1, 0), 0)),
        out_shape=jax.ShapeDtypeStruct((N, D), jnp.float32),
        scratch_shapes=[
            pltpu.VMEM((GRID1, TILE, H), jnp.float32),
            pltpu.VMEM((1, H), jnp.float32),
            pltpu.VMEM((1, H), jnp.float32),
        ],
    )(x, sums, sums, W1, b1.reshape(1, H), gamma.reshape(1, H),
      beta.reshape(1, H), W2, b2.reshape(1, D))


def kernel(x, edge_index, t, W1, b1, gamma, beta, W2, b2):
    src = edge_index[0]
    dst = edge_index[1]
    pad = EPAD - E
    src_pad = jnp.concatenate([src, jnp.full((pad,), N, jnp.int32)])
    dst_pad = jnp.concatenate([dst, jnp.full((pad,), N, jnp.int32)])
    src4 = jnp.stack([src_pad, src_pad + NPAD]).reshape(2, NS, NCHUNK, 1, CH)
    dst4 = jnp.broadcast_to(dst_pad.reshape(1, NS, NCHUNK, 1, CH),
                            (2, NS, NCHUNK, 1, CH))
    sd = jnp.concatenate([src4, dst4], axis=3)
    sd = sd.reshape(2, NS, NGRP, NI, 2, CH)
    zeros = jnp.zeros((RPT, D), jnp.float32)

    tab = _prep(x, t)
    sums = _sc_edge(tab, sd, zeros)
    sums3 = sums.reshape(2, NPAD, D)
    return _mlp(x, sums3, W1, b1, gamma, beta, W2, b2)
